# baseline (speedup 1.0000x reference)
import functools

import jax
import jax.numpy as jnp
from jax import lax
from jax.experimental import pallas as pl
from jax.experimental.pallas import tpu as pltpu
from jax.experimental.pallas import tpu_sc as plsc

HID = 64
WPAD = 128
L = 200
EPS = 1e-12
C1, C2 = 104, 96


def _sc_pool(xs, xt, W, n_side):
    rows_per_w = n_side // 16
    mesh = plsc.VectorSubcoreMesh(core_axis_name="c", subcore_axis_name="s")

    @functools.partial(
        pl.kernel,
        out_type=(
            jax.ShapeDtypeStruct((n_side * HID,), jnp.float32),
            jax.ShapeDtypeStruct((n_side * HID,), jnp.float32),
            jax.ShapeDtypeStruct((n_side * 16,), jnp.float32),
            jax.ShapeDtypeStruct((n_side * 16,), jnp.float32),
        ),
        mesh=mesh,
        scratch_types=[
            pltpu.VMEM((rows_per_w, L), jnp.int32),
            [
                pltpu.VMEM((C1, WPAD), jnp.float32),
                pltpu.VMEM((C2, WPAD), jnp.float32),
                pltpu.VMEM((C1, WPAD), jnp.float32),
                pltpu.VMEM((C2, WPAD), jnp.float32),
            ],
            pltpu.VMEM((rows_per_w * HID,), jnp.float32),
            pltpu.VMEM((rows_per_w * 16,), jnp.float32),
            [pltpu.SemaphoreType.DMA for _ in range(4)],
        ],
        compiler_params=pltpu.CompilerParams(use_tc_tiling_on_sc=False),
    )
    def k(xs_hbm, xt_hbm, w_hbm, os_hbm, ot_hbm, cs_hbm, ct_hbm,
          idx_v, bufs, pooled_v, cnts_v, sems):
        wid = lax.axis_index("s") * 2 + lax.axis_index("c")
        chunk_len = (C1, C2, C1, C2)
        chunk_col = (0, C1, 0, C1)

        def fire(r, b):
            pltpu.async_copy(
                w_hbm.at[idx_v.at[r, pl.ds(chunk_col[b], chunk_len[b])]],
                bufs[b],
                sems[b],
            )

        def wait_buf(b):
            pltpu.make_async_copy(
                w_hbm.at[pl.ds(0, chunk_len[b])], bufs[b], sems[b]
            ).wait()

        lane = lax.iota(jnp.int32, 16)

        def sum_chunk(b, carry):
            zero = jnp.zeros((16,), jnp.float32)

            @pl.loop(0, chunk_len[b], init_carry=carry, unroll=4)
            def _sum(j, c):
                a0, a1, a2, a3 = c
                buf = bufs[b]
                a0 = a0 + buf[j, pl.ds(0, 16)]
                a1 = a1 + buf[j, pl.ds(16, 16)]
                a2 = a2 + buf[j, pl.ds(32, 16)]
                a3 = a3 + buf[j, pl.ds(48, 16)]
                return (a0, a1, a2, a3)

            return _sum

        def count_row(r):
            cnt = jnp.zeros((16,), jnp.float32)
            for j in range(L // 16):
                v = idx_v[r, pl.ds(j * 16, 16)]
                cnt = cnt + jnp.where(v != 0, 1.0, 0.0).astype(jnp.float32)
            v = idx_v[r, pl.ds(L - 16, 16)]
            tail_ok = (v != 0) & (lane >= 16 - L % 16)
            cnt = cnt + jnp.where(tail_ok, 1.0, 0.0).astype(jnp.float32)
            cnts_v[pl.ds(r * 16, 16)] = cnt

        def store_row(r, accs):
            a0, a1, a2, a3 = accs
            ob = r * HID
            pooled_v[pl.ds(ob, 16)] = a0
            pooled_v[pl.ds(ob + 16, 16)] = a1
            pooled_v[pl.ds(ob + 32, 16)] = a2
            pooled_v[pl.ds(ob + 48, 16)] = a3

        def side(x_hbm, out_hbm, cnt_hbm, sw):
            pltpu.sync_copy(x_hbm.at[pl.ds(sw * rows_per_w, rows_per_w), :], idx_v)
            for b in range(4):
                fire(b // 2, b)

            zero = jnp.zeros((16,), jnp.float32)

            @pl.loop(0, rows_per_w // 2)
            def _outer(g):
                for rb in range(2):
                    r = g * 2 + rb
                    count_row(r)
                    wait_buf(2 * rb)
                    accs = sum_chunk(2 * rb, (zero, zero, zero, zero))
                    wait_buf(2 * rb + 1)
                    accs = sum_chunk(2 * rb + 1, accs)
                    store_row(r, accs)
                    for b in (2 * rb, 2 * rb + 1):
                        @pl.when(r + 2 < rows_per_w)
                        def _():
                            fire(r + 2, b)

            pltpu.sync_copy(
                pooled_v,
                out_hbm.at[pl.ds(sw * rows_per_w * HID, rows_per_w * HID)],
            )
            pltpu.sync_copy(
                cnts_v, cnt_hbm.at[pl.ds(sw * rows_per_w * 16, rows_per_w * 16)]
            )

        @pl.when(wid < 16)
        def _():
            side(xs_hbm, os_hbm, cs_hbm, wid)

        @pl.when(wid >= 16)
        def _():
            side(xt_hbm, ot_hbm, ct_hbm, wid - 16)

    return k(xs, xt, W)


def _tc_layernorm(pooled_s, pooled_t, cnts_s, cnts_t, gamma, beta):
    def body(ps_ref, pt_ref, cs_ref, ct_ref, g_ref, b_ref, os_ref, ot_ref):
        g = g_ref[...]
        b = b_ref[...]
        for p_ref, c_ref, o_ref in (
            (ps_ref, cs_ref, os_ref),
            (pt_ref, ct_ref, ot_ref),
        ):
            cnt = jnp.sum(c_ref[...], axis=1, keepdims=True)
            x = p_ref[...] / cnt
            mu = jnp.mean(x, axis=1, keepdims=True)
            d = x - mu
            var = jnp.mean(d * d, axis=1, keepdims=True)
            o_ref[...] = d * lax.rsqrt(var + EPS) * g + b

    n = pooled_s.shape[0]
    return pl.pallas_call(
        body,
        out_shape=(
            jax.ShapeDtypeStruct((n, HID), jnp.float32),
            jax.ShapeDtypeStruct((n, HID), jnp.float32),
        ),
    )(pooled_s, pooled_t, cnts_s, cnts_t,
      gamma.reshape(1, HID), beta.reshape(1, HID))


def kernel(x_s, x_t, W, gamma, beta):
    B = x_s.shape[0]
    W128 = jnp.pad(W, ((0, 0), (0, WPAD - HID)))
    ps, pt, cs, ct = _sc_pool(x_s.astype(jnp.int32), x_t.astype(jnp.int32), W128, B)
    out_s, out_t = _tc_layernorm(
        ps.reshape(B, HID),
        pt.reshape(B, HID),
        cs.reshape(B, 16),
        ct.reshape(B, 16),
        gamma,
        beta,
    )
    return out_s, out_t
